# baseline (speedup 1.0000x reference)
import jax
import jax.numpy as jnp
from jax.experimental import pallas as pl

VOCAB = 32000
D = 1024
MAXP = 16
TEMP = 1.0
NB = 3200


def _h_kernel(a_ref, wh_ref, o_ref):
    o_ref[...] = jnp.tanh(jnp.dot(a_ref[...], wh_ref[...],
                                  preferred_element_type=jnp.float32))


def _logits_kernel(h_ref, w_ref, o_ref):
    o_ref[...] = jnp.dot(h_ref[...], w_ref[...],
                         preferred_element_type=jnp.float32)


def _pallas_h(a, W_h):
    B = a.shape[0]
    return pl.pallas_call(
        _h_kernel,
        out_shape=jax.ShapeDtypeStruct((B, D), jnp.float32),
    )(a, W_h)


def _pallas_logits(h, W_out):
    B = h.shape[0]
    return pl.pallas_call(
        _logits_kernel,
        grid=(VOCAB // NB,),
        in_specs=[pl.BlockSpec((B, D), lambda i: (0, 0)),
                  pl.BlockSpec((D, NB), lambda i: (0, i))],
        out_specs=pl.BlockSpec((B, NB), lambda i: (0, i)),
        out_shape=jax.ShapeDtypeStruct((B, VOCAB), jnp.float32),
    )(h, W_out)


def kernel(X, E, W_h, W_out, rand_u):
    Bn = X.shape[0]
    ctx = jnp.mean(jnp.take(E, X, axis=0), axis=1)
    Y = jnp.ones((Bn, 1), dtype=jnp.int32)
    log_probabilities = jnp.zeros((Bn,), dtype=jnp.float32)
    for i in range(MAXP):
        a = jnp.take(E, Y[:, -1], axis=0) + ctx
        h = _pallas_h(a, W_h)
        next_log_probabilities = _pallas_logits(h, W_out)
        next_probabilities = jax.nn.softmax(next_log_probabilities / TEMP, axis=1)
        random = rand_u[i]
        next_chars = jnp.sum(jnp.cumsum(next_probabilities, axis=1) < random,
                             axis=1, keepdims=True).astype(jnp.int32)
        next_chars = jnp.clip(next_chars, 0, VOCAB - 1)
        lp = jax.nn.log_softmax(next_log_probabilities / TEMP, axis=1)
        log_probabilities = log_probabilities + jnp.take_along_axis(lp, next_chars, axis=1)[:, 0]
        Y = jnp.concatenate([Y, next_chars], axis=1)
    return Y, log_probabilities
